# baseline (speedup 1.0000x reference)
import functools

import jax
import jax.numpy as jnp
from jax import lax
from jax.experimental import pallas as pl
from jax.experimental.pallas import tpu as pltpu
from jax.experimental.pallas import tpu_sc as plsc

DIM = 64
GAMMA = 12.0
CHUNK = 1024


@functools.cache
def _make_tc_transpose(n_entity: int):
    grid = (n_entity // 2 + CHUNK - 1) // CHUNK
    n_rows = grid * CHUNK
    last_in_block = (n_entity - 1) // CHUNK

    def hi_map(c):
        return (0, jnp.minimum(c + grid, last_in_block))

    def body(elo_ref, ehi_ref, rlo_ref, rhi_ref, eo_ref, ro_ref):
        eye = jnp.eye(DIM, dtype=jnp.float32)
        for lo, hi, dst in ((elo_ref, ehi_ref, eo_ref),
                            (rlo_ref, rhi_ref, ro_ref)):
            dst[:, 0:DIM] = lax.dot_general(
                lo[...], eye, (((0,), (0,)), ((), ())),
                preferred_element_type=jnp.float32)
            dst[:, DIM:2 * DIM] = lax.dot_general(
                hi[...], eye, (((0,), (0,)), ((), ())),
                preferred_element_type=jnp.float32)

    return pl.pallas_call(
        body,
        grid=(grid,),
        in_specs=[
            pl.BlockSpec((DIM, CHUNK), lambda c: (0, c)),
            pl.BlockSpec((DIM, CHUNK), hi_map),
            pl.BlockSpec((DIM, CHUNK), lambda c: (0, c)),
            pl.BlockSpec((DIM, CHUNK), hi_map),
        ],
        out_specs=[
            pl.BlockSpec((CHUNK, 2 * DIM), lambda c: (c, 0)),
            pl.BlockSpec((CHUNK, 2 * DIM), lambda c: (c, 0)),
        ],
        out_shape=[
            jax.ShapeDtypeStruct((n_rows, 2 * DIM), jnp.float32),
            jax.ShapeDtypeStruct((n_rows, 2 * DIM), jnp.float32),
        ],
    )


@functools.cache
def _make_phase_b(B: int, n_rows: int):
    info = plsc.get_sparse_core_info()
    NC, NS, L = info.num_cores, info.num_subcores, info.num_lanes
    NW = NC * NS
    BW = B // NW
    NG = BW // L
    mesh = plsc.VectorSubcoreMesh(core_axis_name="c", subcore_axis_name="s")

    @functools.partial(
        pl.kernel,
        mesh=mesh,
        compiler_params=pltpu.CompilerParams(needs_layout_passes=False),
        out_type=jax.ShapeDtypeStruct((B,), jnp.float32),
        scratch_types=[
            pltpu.VMEM((BW,), jnp.int32),
            pltpu.VMEM((BW,), jnp.int32),
            pltpu.VMEM((BW,), jnp.int32),
            pltpu.VMEM((L, 2 * DIM), jnp.float32),
            pltpu.VMEM((L, 2 * DIM), jnp.float32),
            pltpu.VMEM((L, 2 * DIM), jnp.float32),
            pltpu.VMEM((BW,), jnp.float32),
            pltpu.SemaphoreType.DMA,
        ],
    )
    def kb(hidx_hbm, ridx_hbm, tidx_hbm, ent_hbm, rel_hbm, out_hbm,
           hidx_v, ridx_v, tidx_v, h_v, r_v, t_v, out_v, sem):
        wid = lax.axis_index("s") * NC + lax.axis_index("c")
        base = wid * BW
        pltpu.sync_copy(hidx_hbm.at[pl.ds(base, BW)], hidx_v)
        pltpu.sync_copy(ridx_hbm.at[pl.ds(base, BW)], ridx_v)
        pltpu.sync_copy(tidx_hbm.at[pl.ds(base, BW)], tidx_v)

        lanes = lax.iota(jnp.int32, L)

        def group(g, carry):
            sl = pl.ds(g * L, L)
            his = hidx_v[sl]
            ris = ridx_v[sl]
            tis = tidx_v[sl]
            hrow = jnp.where(his < n_rows, his, his - n_rows)
            rrow = jnp.where(ris < n_rows, ris, ris - n_rows)
            trow = jnp.where(tis < n_rows, tis, tis - n_rows)
            cps = [
                pltpu.async_copy(ent_hbm.at[hrow], h_v, sem),
                pltpu.async_copy(rel_hbm.at[rrow], r_v, sem),
                pltpu.async_copy(ent_hbm.at[trow], t_v, sem),
            ]
            for c in cps:
                c.wait()

            hoff = jnp.where(his < n_rows, 0, DIM)
            roff = jnp.where(ris < n_rows, 0, DIM)
            toff = jnp.where(tis < n_rows, 0, DIM)
            acc = jnp.zeros((L,), jnp.float32)
            for d in range(DIM):
                h = plsc.load_gather(h_v, [lanes, hoff + d])
                r = plsc.load_gather(r_v, [lanes, roff + d])
                t = plsc.load_gather(t_v, [lanes, toff + d])
                acc = acc + jnp.abs(h + r - t)
            out_v[sl] = GAMMA - acc
            return carry

        lax.fori_loop(0, NG, group, 0)
        pltpu.sync_copy(out_v, out_hbm.at[pl.ds(base, BW)])

    return kb


@jax.jit
def kernel(sample, entity_embedding, relation_embedding):
    B = sample.shape[0]
    n = entity_embedding.shape[0]
    hidx = sample[:, 0]
    ridx = sample[:, 1]
    tidx = sample[:, 2]
    ent_t = entity_embedding.T
    rel_t = relation_embedding.T
    ent_stage, rel_stage = _make_tc_transpose(n)(ent_t, ent_t, rel_t, rel_t)
    score = _make_phase_b(B, ent_stage.shape[0])(
        hidx, ridx, tidx, ent_stage, rel_stage)
    return score.reshape(B, 1)


# speedup vs baseline: 1.4367x; 1.4367x over previous
import functools

import jax
import jax.numpy as jnp
from jax import lax
from jax.experimental import pallas as pl
from jax.experimental.pallas import tpu as pltpu
from jax.experimental.pallas import tpu_sc as plsc

DIM = 64
GAMMA = 12.0
CHUNK = 4096


@functools.cache
def _make_tc_transpose(n_entity: int):
    grid = (n_entity // 2 + CHUNK - 1) // CHUNK
    n_rows = grid * CHUNK
    last_in_block = (n_entity - 1) // CHUNK

    def hi_map(c):
        return (0, jnp.minimum(c + grid, last_in_block))

    def body(elo_ref, ehi_ref, rlo_ref, rhi_ref, eo_ref, ro_ref):
        for lo, hi, dst in ((elo_ref, ehi_ref, eo_ref),
                            (rlo_ref, rhi_ref, ro_ref)):
            dst[:, 0:DIM] = lo[...].T
            dst[:, DIM:2 * DIM] = hi[...].T

    return pl.pallas_call(
        body,
        grid=(grid,),
        in_specs=[
            pl.BlockSpec((DIM, CHUNK), lambda c: (0, c)),
            pl.BlockSpec((DIM, CHUNK), hi_map),
            pl.BlockSpec((DIM, CHUNK), lambda c: (0, c)),
            pl.BlockSpec((DIM, CHUNK), hi_map),
        ],
        out_specs=[
            pl.BlockSpec((CHUNK, 2 * DIM), lambda c: (c, 0)),
            pl.BlockSpec((CHUNK, 2 * DIM), lambda c: (c, 0)),
        ],
        out_shape=[
            jax.ShapeDtypeStruct((n_rows, 2 * DIM), jnp.float32),
            jax.ShapeDtypeStruct((n_rows, 2 * DIM), jnp.float32),
        ],
    )


@functools.cache
def _make_phase_b(B: int, n_rows: int):
    info = plsc.get_sparse_core_info()
    NC, NS, L = info.num_cores, info.num_subcores, info.num_lanes
    NW = NC * NS
    BW = B // NW
    NG = BW // L
    mesh = plsc.VectorSubcoreMesh(core_axis_name="c", subcore_axis_name="s")

    @functools.partial(
        pl.kernel,
        mesh=mesh,
        compiler_params=pltpu.CompilerParams(needs_layout_passes=False),
        out_type=jax.ShapeDtypeStruct((B,), jnp.float32),
        scratch_types=[
            pltpu.VMEM((BW,), jnp.int32),
            pltpu.VMEM((BW,), jnp.int32),
            pltpu.VMEM((BW,), jnp.int32),
            pltpu.VMEM((L, 2 * DIM), jnp.float32),
            pltpu.VMEM((L, 2 * DIM), jnp.float32),
            pltpu.VMEM((L, 2 * DIM), jnp.float32),
            pltpu.VMEM((BW,), jnp.float32),
            pltpu.SemaphoreType.DMA,
        ],
    )
    def kb(hidx_hbm, ridx_hbm, tidx_hbm, ent_hbm, rel_hbm, out_hbm,
           hidx_v, ridx_v, tidx_v, h_v, r_v, t_v, out_v, sem):
        wid = lax.axis_index("s") * NC + lax.axis_index("c")
        base = wid * BW
        pltpu.sync_copy(hidx_hbm.at[pl.ds(base, BW)], hidx_v)
        pltpu.sync_copy(ridx_hbm.at[pl.ds(base, BW)], ridx_v)
        pltpu.sync_copy(tidx_hbm.at[pl.ds(base, BW)], tidx_v)

        lanes = lax.iota(jnp.int32, L)

        def group(g, carry):
            sl = pl.ds(g * L, L)
            his = hidx_v[sl]
            ris = ridx_v[sl]
            tis = tidx_v[sl]
            hrow = jnp.where(his < n_rows, his, his - n_rows)
            rrow = jnp.where(ris < n_rows, ris, ris - n_rows)
            trow = jnp.where(tis < n_rows, tis, tis - n_rows)
            cps = [
                pltpu.async_copy(ent_hbm.at[hrow], h_v, sem),
                pltpu.async_copy(rel_hbm.at[rrow], r_v, sem),
                pltpu.async_copy(ent_hbm.at[trow], t_v, sem),
            ]
            for c in cps:
                c.wait()

            hoff = jnp.where(his < n_rows, 0, DIM)
            roff = jnp.where(ris < n_rows, 0, DIM)
            toff = jnp.where(tis < n_rows, 0, DIM)
            acc = jnp.zeros((L,), jnp.float32)
            for d in range(DIM):
                h = plsc.load_gather(h_v, [lanes, hoff + d])
                r = plsc.load_gather(r_v, [lanes, roff + d])
                t = plsc.load_gather(t_v, [lanes, toff + d])
                acc = acc + jnp.abs(h + r - t)
            out_v[sl] = GAMMA - acc
            return carry

        lax.fori_loop(0, NG, group, 0)
        pltpu.sync_copy(out_v, out_hbm.at[pl.ds(base, BW)])

    return kb


@jax.jit
def kernel(sample, entity_embedding, relation_embedding):
    B = sample.shape[0]
    n = entity_embedding.shape[0]
    hidx = sample[:, 0]
    ridx = sample[:, 1]
    tidx = sample[:, 2]
    ent_t = entity_embedding.T
    rel_t = relation_embedding.T
    ent_stage, rel_stage = _make_tc_transpose(n)(ent_t, ent_t, rel_t, rel_t)
    score = _make_phase_b(B, ent_stage.shape[0])(
        hidx, ridx, tidx, ent_stage, rel_stage)
    return score.reshape(B, 1)


# speedup vs baseline: 1.4530x; 1.0113x over previous
import functools

import jax
import jax.numpy as jnp
from jax import lax
from jax.experimental import pallas as pl
from jax.experimental.pallas import tpu as pltpu
from jax.experimental.pallas import tpu_sc as plsc

DIM = 64
GAMMA = 12.0
CHUNK = 8192


@functools.cache
def _make_tc_transpose(n_entity: int):
    grid = (n_entity // 2 + CHUNK - 1) // CHUNK
    n_rows = grid * CHUNK
    last_in_block = (n_entity - 1) // CHUNK

    def hi_map(c):
        return (0, jnp.minimum(c + grid, last_in_block))

    def body(elo_ref, ehi_ref, rlo_ref, rhi_ref, eo_ref, ro_ref):
        for lo, hi, dst in ((elo_ref, ehi_ref, eo_ref),
                            (rlo_ref, rhi_ref, ro_ref)):
            dst[:, 0:DIM] = lo[...].T
            dst[:, DIM:2 * DIM] = hi[...].T

    return pl.pallas_call(
        body,
        grid=(grid,),
        in_specs=[
            pl.BlockSpec((DIM, CHUNK), lambda c: (0, c)),
            pl.BlockSpec((DIM, CHUNK), hi_map),
            pl.BlockSpec((DIM, CHUNK), lambda c: (0, c)),
            pl.BlockSpec((DIM, CHUNK), hi_map),
        ],
        out_specs=[
            pl.BlockSpec((CHUNK, 2 * DIM), lambda c: (c, 0)),
            pl.BlockSpec((CHUNK, 2 * DIM), lambda c: (c, 0)),
        ],
        out_shape=[
            jax.ShapeDtypeStruct((n_rows, 2 * DIM), jnp.float32),
            jax.ShapeDtypeStruct((n_rows, 2 * DIM), jnp.float32),
        ],
    )


@functools.cache
def _make_phase_b(B: int, n_rows: int):
    info = plsc.get_sparse_core_info()
    NC, NS, L = info.num_cores, info.num_subcores, info.num_lanes
    NW = NC * NS
    BW = B // NW
    NG = BW // L
    mesh = plsc.VectorSubcoreMesh(core_axis_name="c", subcore_axis_name="s")

    @functools.partial(
        pl.kernel,
        mesh=mesh,
        compiler_params=pltpu.CompilerParams(needs_layout_passes=False),
        out_type=jax.ShapeDtypeStruct((B,), jnp.float32),
        scratch_types=[
            pltpu.VMEM((BW,), jnp.int32),
            pltpu.VMEM((BW,), jnp.int32),
            pltpu.VMEM((BW,), jnp.int32),
            pltpu.VMEM((L, 2 * DIM), jnp.float32),
            pltpu.VMEM((L, 2 * DIM), jnp.float32),
            pltpu.VMEM((L, 2 * DIM), jnp.float32),
            pltpu.VMEM((BW,), jnp.float32),
            pltpu.SemaphoreType.DMA,
        ],
    )
    def kb(hidx_hbm, ridx_hbm, tidx_hbm, ent_hbm, rel_hbm, out_hbm,
           hidx_v, ridx_v, tidx_v, h_v, r_v, t_v, out_v, sem):
        wid = lax.axis_index("s") * NC + lax.axis_index("c")
        base = wid * BW
        pltpu.sync_copy(hidx_hbm.at[pl.ds(base, BW)], hidx_v)
        pltpu.sync_copy(ridx_hbm.at[pl.ds(base, BW)], ridx_v)
        pltpu.sync_copy(tidx_hbm.at[pl.ds(base, BW)], tidx_v)

        lanes = lax.iota(jnp.int32, L)

        def group(g, carry):
            sl = pl.ds(g * L, L)
            his = hidx_v[sl]
            ris = ridx_v[sl]
            tis = tidx_v[sl]
            hrow = jnp.where(his < n_rows, his, his - n_rows)
            rrow = jnp.where(ris < n_rows, ris, ris - n_rows)
            trow = jnp.where(tis < n_rows, tis, tis - n_rows)
            cps = [
                pltpu.async_copy(ent_hbm.at[hrow], h_v, sem),
                pltpu.async_copy(rel_hbm.at[rrow], r_v, sem),
                pltpu.async_copy(ent_hbm.at[trow], t_v, sem),
            ]
            for c in cps:
                c.wait()

            hoff = jnp.where(his < n_rows, 0, DIM)
            roff = jnp.where(ris < n_rows, 0, DIM)
            toff = jnp.where(tis < n_rows, 0, DIM)
            acc = jnp.zeros((L,), jnp.float32)
            for d in range(DIM):
                h = plsc.load_gather(h_v, [lanes, hoff + d])
                r = plsc.load_gather(r_v, [lanes, roff + d])
                t = plsc.load_gather(t_v, [lanes, toff + d])
                acc = acc + jnp.abs(h + r - t)
            out_v[sl] = GAMMA - acc
            return carry

        lax.fori_loop(0, NG, group, 0)
        pltpu.sync_copy(out_v, out_hbm.at[pl.ds(base, BW)])

    return kb


@jax.jit
def kernel(sample, entity_embedding, relation_embedding):
    B = sample.shape[0]
    n = entity_embedding.shape[0]
    hidx = sample[:, 0]
    ridx = sample[:, 1]
    tidx = sample[:, 2]
    ent_t = entity_embedding.T
    rel_t = relation_embedding.T
    ent_stage, rel_stage = _make_tc_transpose(n)(ent_t, ent_t, rel_t, rel_t)
    score = _make_phase_b(B, ent_stage.shape[0])(
        hidx, ridx, tidx, ent_stage, rel_stage)
    return score.reshape(B, 1)


# speedup vs baseline: 1.5361x; 1.0572x over previous
import functools

import jax
import jax.numpy as jnp
from jax import lax
from jax.experimental import pallas as pl
from jax.experimental.pallas import tpu as pltpu
from jax.experimental.pallas import tpu_sc as plsc

DIM = 64
GAMMA = 12.0
CHUNK = 8192


@functools.cache
def _make_tc_transpose(n_entity: int):
    grid = (n_entity // 2 + CHUNK - 1) // CHUNK
    n_rows = grid * CHUNK
    last_in_block = (n_entity - 1) // CHUNK

    def hi_map(c):
        return (0, jnp.minimum(c + grid, last_in_block))

    def body(elo_ref, ehi_ref, rlo_ref, rhi_ref, eo_ref, ro_ref):
        for lo, hi, dst in ((elo_ref, ehi_ref, eo_ref),
                            (rlo_ref, rhi_ref, ro_ref)):
            dst[:, 0:DIM] = lo[...].T
            dst[:, DIM:2 * DIM] = hi[...].T

    return pl.pallas_call(
        body,
        grid=(grid,),
        in_specs=[
            pl.BlockSpec((DIM, CHUNK), lambda c: (0, c)),
            pl.BlockSpec((DIM, CHUNK), hi_map),
            pl.BlockSpec((DIM, CHUNK), lambda c: (0, c)),
            pl.BlockSpec((DIM, CHUNK), hi_map),
        ],
        out_specs=[
            pl.BlockSpec((CHUNK, 2 * DIM), lambda c: (c, 0)),
            pl.BlockSpec((CHUNK, 2 * DIM), lambda c: (c, 0)),
        ],
        out_shape=[
            jax.ShapeDtypeStruct((n_rows, 2 * DIM), jnp.float32),
            jax.ShapeDtypeStruct((n_rows, 2 * DIM), jnp.float32),
        ],
    )


@functools.cache
def _make_phase_b(B: int, n_rows: int):
    info = plsc.get_sparse_core_info()
    NC, NS, L = info.num_cores, info.num_subcores, info.num_lanes
    NW = NC * NS
    BW = B // NW
    NG = BW // L
    mesh = plsc.VectorSubcoreMesh(core_axis_name="c", subcore_axis_name="s")

    @functools.partial(
        pl.kernel,
        mesh=mesh,
        compiler_params=pltpu.CompilerParams(needs_layout_passes=False),
        out_type=jax.ShapeDtypeStruct((B,), jnp.float32),
        scratch_types=[
            pltpu.VMEM((BW,), jnp.int32),
            pltpu.VMEM((BW,), jnp.int32),
            pltpu.VMEM((BW,), jnp.int32),
            pltpu.VMEM((2, L, 2 * DIM), jnp.float32),
            pltpu.VMEM((2, L, 2 * DIM), jnp.float32),
            pltpu.VMEM((2, L, 2 * DIM), jnp.float32),
            pltpu.VMEM((BW,), jnp.float32),
            pltpu.SemaphoreType.DMA,
            pltpu.SemaphoreType.DMA,
        ],
    )
    def kb(hidx_hbm, ridx_hbm, tidx_hbm, ent_hbm, rel_hbm, out_hbm,
           hidx_v, ridx_v, tidx_v, h_v, r_v, t_v, out_v, sem0, sem1):
        wid = lax.axis_index("s") * NC + lax.axis_index("c")
        base = wid * BW
        pltpu.sync_copy(hidx_hbm.at[pl.ds(base, BW)], hidx_v)
        pltpu.sync_copy(ridx_hbm.at[pl.ds(base, BW)], ridx_v)
        pltpu.sync_copy(tidx_hbm.at[pl.ds(base, BW)], tidx_v)

        lanes = lax.iota(jnp.int32, L)
        dummy = ent_hbm.at[pl.ds(0, L)]

        def idxs(g):
            sl = pl.ds(g * L, L)
            return hidx_v[sl], ridx_v[sl], tidx_v[sl]

        def fire(g, slot, sem):
            his, ris, tis = idxs(g)
            hrow = jnp.where(his < n_rows, his, his - n_rows)
            rrow = jnp.where(ris < n_rows, ris, ris - n_rows)
            trow = jnp.where(tis < n_rows, tis, tis - n_rows)
            pltpu.async_copy(ent_hbm.at[hrow], h_v.at[slot], sem)
            pltpu.async_copy(rel_hbm.at[rrow], r_v.at[slot], sem)
            pltpu.async_copy(ent_hbm.at[trow], t_v.at[slot], sem)

        def drain(slot, sem):
            pltpu.make_async_copy(dummy, h_v.at[slot], sem).wait()
            pltpu.make_async_copy(dummy, r_v.at[slot], sem).wait()
            pltpu.make_async_copy(dummy, t_v.at[slot], sem).wait()

        def compute(g, slot):
            his, ris, tis = idxs(g)
            hoff = jnp.where(his < n_rows, 0, DIM)
            roff = jnp.where(ris < n_rows, 0, DIM)
            toff = jnp.where(tis < n_rows, 0, DIM)
            acc = jnp.zeros((L,), jnp.float32)
            for d in range(DIM):
                h = plsc.load_gather(h_v.at[slot], [lanes, hoff + d])
                r = plsc.load_gather(r_v.at[slot], [lanes, roff + d])
                t = plsc.load_gather(t_v.at[slot], [lanes, toff + d])
                acc = acc + jnp.abs(h + r - t)
            out_v[pl.ds(g * L, L)] = GAMMA - acc

        fire(0, 0, sem0)

        def step(i, carry):
            g0 = 2 * i

            @pl.when(g0 + 1 < NG)
            def _():
                fire(g0 + 1, 1, sem1)

            drain(0, sem0)
            compute(g0, 0)

            @pl.when(g0 + 2 < NG)
            def _():
                fire(g0 + 2, 0, sem0)

            @pl.when(g0 + 1 < NG)
            def _():
                drain(1, sem1)
                compute(g0 + 1, 1)

            return carry

        lax.fori_loop(0, (NG + 1) // 2, step, 0)
        pltpu.sync_copy(out_v, out_hbm.at[pl.ds(base, BW)])

    return kb


@jax.jit
def kernel(sample, entity_embedding, relation_embedding):
    B = sample.shape[0]
    n = entity_embedding.shape[0]
    hidx = sample[:, 0]
    ridx = sample[:, 1]
    tidx = sample[:, 2]
    ent_t = entity_embedding.T
    rel_t = relation_embedding.T
    ent_stage, rel_stage = _make_tc_transpose(n)(ent_t, ent_t, rel_t, rel_t)
    score = _make_phase_b(B, ent_stage.shape[0])(
        hidx, ridx, tidx, ent_stage, rel_stage)
    return score.reshape(B, 1)
